# initial kernel scaffold (unmeasured)
import jax
import jax.numpy as jnp
from jax import lax
from jax.experimental import pallas as pl
from jax.experimental.pallas import tpu as pltpu


def kernel(
    x,
):
    def body(*refs):
        pass

    out_shape = jax.ShapeDtypeStruct(..., jnp.float32)
    return pl.pallas_call(body, out_shape=out_shape)(...)


# baseline (device time: 19212 ns/iter reference)
import jax
import jax.numpy as jnp
from jax import lax
from jax.experimental import pallas as pl
from jax.experimental.pallas import tpu as pltpu

N_DEV = 4


def kernel(x):
    m_rows, n_cols = x.shape

    def body(x_ref, out_ref, gather_ref, send_sems, recv_sems):
        my = lax.axis_index("i")
        left = lax.rem(my + N_DEV - 1, N_DEV)
        right = lax.rem(my + 1, N_DEV)

        barrier_sem = pltpu.get_barrier_semaphore()
        for nbr in [left, right]:
            pl.semaphore_signal(
                barrier_sem, inc=1,
                device_id=(nbr,), device_id_type=pl.DeviceIdType.MESH,
            )
        pl.semaphore_wait(barrier_sem, 2)

        xv = x_ref[:, :].astype(jnp.float32)
        m = jnp.max(xv, axis=1, keepdims=True)
        e = jnp.exp(xv - m)
        s = jnp.sum(e, axis=1, keepdims=True)

        stats = jnp.concatenate([m, s], axis=1)
        gather_ref[pl.ds(my, 1)] = stats[None, :, :]

        for h in range(N_DEV - 1):
            slot = lax.rem(my - h + N_DEV, N_DEV)
            rdma = pltpu.make_async_remote_copy(
                src_ref=gather_ref.at[slot],
                dst_ref=gather_ref.at[slot],
                send_sem=send_sems.at[h],
                recv_sem=recv_sems.at[h],
                device_id=(right,),
                device_id_type=pl.DeviceIdType.MESH,
            )
            rdma.start()
            rdma.wait()

        g = gather_ref[:, :, :]
        m_all = g[:, :, 0:1]
        s_all = g[:, :, 1:2]
        gmax = jnp.max(m_all, axis=0)
        gsum = jnp.sum(s_all * jnp.exp(m_all - gmax[None]), axis=0)

        out_ref[:, :] = e * (jnp.exp(m - gmax) / gsum)

    return pl.pallas_call(
        body,
        out_shape=jax.ShapeDtypeStruct((m_rows, n_cols), jnp.float32),
        in_specs=[pl.BlockSpec(memory_space=pltpu.VMEM)],
        out_specs=pl.BlockSpec(memory_space=pltpu.VMEM),
        scratch_shapes=[
            pltpu.VMEM((N_DEV, m_rows, 2), jnp.float32),
            pltpu.SemaphoreType.DMA((N_DEV - 1,)),
            pltpu.SemaphoreType.DMA((N_DEV - 1,)),
        ],
        compiler_params=pltpu.CompilerParams(collective_id=0),
    )(x)


# device time: 13124 ns/iter; 1.4639x vs baseline; 1.4639x over previous
import jax
import jax.numpy as jnp
from jax import lax
from jax.experimental import pallas as pl
from jax.experimental.pallas import tpu as pltpu

N_DEV = 4


def kernel(x):
    m_rows, n_cols = x.shape

    def body(x_ref, out_ref, gather_ref, send_sems, recv_sems):
        my = lax.axis_index("i")

        xv = x_ref[:, :].astype(jnp.float32)
        m = jnp.max(xv, axis=1, keepdims=True)
        e = jnp.exp(xv - m)
        s = jnp.sum(e, axis=1, keepdims=True)
        stats = jnp.concatenate([m, s], axis=1)
        gather_ref[pl.ds(my, 1)] = stats[None, :, :]

        barrier_sem = pltpu.get_barrier_semaphore()
        for d in range(1, N_DEV):
            peer = lax.rem(my + d, N_DEV)
            pl.semaphore_signal(
                barrier_sem, inc=1,
                device_id=(peer,), device_id_type=pl.DeviceIdType.MESH,
            )
        pl.semaphore_wait(barrier_sem, N_DEV - 1)

        sends = []
        for d in range(1, N_DEV):
            peer = lax.rem(my + d, N_DEV)
            rdma = pltpu.make_async_remote_copy(
                src_ref=gather_ref.at[my],
                dst_ref=gather_ref.at[my],
                send_sem=send_sems.at[d - 1],
                recv_sem=recv_sems.at[my],
                device_id=(peer,),
                device_id_type=pl.DeviceIdType.MESH,
            )
            rdma.start()
            sends.append(rdma)

        for d in range(1, N_DEV):
            peer = lax.rem(my + d, N_DEV)
            recv = pltpu.make_async_remote_copy(
                src_ref=gather_ref.at[peer],
                dst_ref=gather_ref.at[peer],
                send_sem=send_sems.at[d - 1],
                recv_sem=recv_sems.at[peer],
                device_id=(peer,),
                device_id_type=pl.DeviceIdType.MESH,
            )
            recv.wait_recv()
        for rdma in sends:
            rdma.wait_send()

        g = gather_ref[:, :, :]
        m_all = g[:, :, 0:1]
        s_all = g[:, :, 1:2]
        gmax = jnp.max(m_all, axis=0)
        gsum = jnp.sum(s_all * jnp.exp(m_all - gmax[None]), axis=0)

        out_ref[:, :] = e * (jnp.exp(m - gmax) / gsum)

    return pl.pallas_call(
        body,
        out_shape=jax.ShapeDtypeStruct((m_rows, n_cols), jnp.float32),
        in_specs=[pl.BlockSpec(memory_space=pltpu.VMEM)],
        out_specs=pl.BlockSpec(memory_space=pltpu.VMEM),
        scratch_shapes=[
            pltpu.VMEM((N_DEV, m_rows, 2), jnp.float32),
            pltpu.SemaphoreType.DMA((N_DEV - 1,)),
            pltpu.SemaphoreType.DMA((N_DEV,)),
        ],
        compiler_params=pltpu.CompilerParams(collective_id=0),
    )(x)


# device time: 2411 ns/iter; 7.9685x vs baseline; 5.4434x over previous
import jax
import jax.numpy as jnp
from jax import lax
from jax.experimental import pallas as pl
from jax.experimental.pallas import tpu as pltpu

N_DEV = 4


def kernel(x):
    m_rows, n_cols = x.shape

    def body(x_ref, out_ref, gather_ref, send_sems, recv_sems):
        my = lax.axis_index("i")

        xv = x_ref[:, :].astype(jnp.float32)
        m = jnp.max(xv, axis=1, keepdims=True)
        e = jnp.exp(xv - m)
        s = jnp.sum(e, axis=1, keepdims=True)
        stats_t = jnp.stack([m[:, 0], s[:, 0]], axis=0)
        gather_ref[pl.ds(my, 1)] = stats_t[None, :, :]

        barrier_sem = pltpu.get_barrier_semaphore()
        for d in range(1, N_DEV):
            peer = lax.rem(my + d, N_DEV)
            pl.semaphore_signal(
                barrier_sem, inc=1,
                device_id=(peer,), device_id_type=pl.DeviceIdType.MESH,
            )
        pl.semaphore_wait(barrier_sem, N_DEV - 1)

        sends = []
        for d in range(1, N_DEV):
            peer = lax.rem(my + d, N_DEV)
            rdma = pltpu.make_async_remote_copy(
                src_ref=gather_ref.at[my],
                dst_ref=gather_ref.at[my],
                send_sem=send_sems.at[d - 1],
                recv_sem=recv_sems.at[my],
                device_id=(peer,),
                device_id_type=pl.DeviceIdType.MESH,
            )
            rdma.start()
            sends.append(rdma)

        for d in range(1, N_DEV):
            peer = lax.rem(my + d, N_DEV)
            recv = pltpu.make_async_remote_copy(
                src_ref=gather_ref.at[peer],
                dst_ref=gather_ref.at[peer],
                send_sem=send_sems.at[d - 1],
                recv_sem=recv_sems.at[peer],
                device_id=(peer,),
                device_id_type=pl.DeviceIdType.MESH,
            )
            recv.wait_recv()
        for rdma in sends:
            rdma.wait_send()

        g = gather_ref[:, :, :]
        m_all = g[:, 0, :]
        s_all = g[:, 1, :]
        gmax = jnp.max(m_all, axis=0)
        gsum = jnp.sum(s_all * jnp.exp(m_all - gmax[None, :]), axis=0)
        scale = jnp.exp(stats_t[0] - gmax) / gsum

        out_ref[:, :] = e * scale[:, None]

    return pl.pallas_call(
        body,
        out_shape=jax.ShapeDtypeStruct((m_rows, n_cols), jnp.float32),
        in_specs=[pl.BlockSpec(memory_space=pltpu.VMEM)],
        out_specs=pl.BlockSpec(memory_space=pltpu.VMEM),
        scratch_shapes=[
            pltpu.VMEM((N_DEV, 2, m_rows), jnp.float32),
            pltpu.SemaphoreType.DMA((N_DEV - 1,)),
            pltpu.SemaphoreType.DMA((N_DEV,)),
        ],
        compiler_params=pltpu.CompilerParams(collective_id=0),
    )(x)


# device time: 2143 ns/iter; 8.9650x vs baseline; 1.1251x over previous
import jax
import jax.numpy as jnp
from jax import lax
from jax.experimental import pallas as pl
from jax.experimental.pallas import tpu as pltpu

N_DEV = 4


def kernel(x):
    m_rows, n_cols = x.shape

    def body(x_ref, out_ref, gather_ref, send_sems, recv_sems):
        my = lax.axis_index("i")

        barrier_sem = pltpu.get_barrier_semaphore()
        for d in range(1, N_DEV):
            peer = lax.rem(my + d, N_DEV)
            pl.semaphore_signal(
                barrier_sem, inc=1,
                device_id=(peer,), device_id_type=pl.DeviceIdType.MESH,
            )

        xv = x_ref[:, :].astype(jnp.float32)
        m = jnp.max(xv, axis=1, keepdims=True)
        e = jnp.exp(xv - m)
        s = jnp.sum(e, axis=1, keepdims=True)
        stats_t = jnp.stack([m[:, 0], s[:, 0]], axis=0)
        gather_ref[pl.ds(my, 1)] = stats_t[None, :, :]

        pl.semaphore_wait(barrier_sem, N_DEV - 1)

        sends = []
        for d in range(1, N_DEV):
            peer = lax.rem(my + d, N_DEV)
            rdma = pltpu.make_async_remote_copy(
                src_ref=gather_ref.at[my],
                dst_ref=gather_ref.at[my],
                send_sem=send_sems.at[d - 1],
                recv_sem=recv_sems.at[my],
                device_id=(peer,),
                device_id_type=pl.DeviceIdType.MESH,
            )
            rdma.start()
            sends.append(rdma)

        for d in range(1, N_DEV):
            peer = lax.rem(my + d, N_DEV)
            recv = pltpu.make_async_remote_copy(
                src_ref=gather_ref.at[peer],
                dst_ref=gather_ref.at[peer],
                send_sem=send_sems.at[d - 1],
                recv_sem=recv_sems.at[peer],
                device_id=(peer,),
                device_id_type=pl.DeviceIdType.MESH,
            )
            recv.wait_recv()
        for rdma in sends:
            rdma.wait_send()

        g = gather_ref[:, :, :]
        m_all = g[:, 0, :]
        s_all = g[:, 1, :]
        gmax = jnp.max(m_all, axis=0)
        gsum = jnp.sum(s_all * jnp.exp(m_all - gmax[None, :]), axis=0)
        scale = jnp.exp(stats_t[0] - gmax) / gsum

        out_ref[:, :] = e * scale[:, None]

    return pl.pallas_call(
        body,
        out_shape=jax.ShapeDtypeStruct((m_rows, n_cols), jnp.float32),
        in_specs=[pl.BlockSpec(memory_space=pltpu.VMEM)],
        out_specs=pl.BlockSpec(memory_space=pltpu.VMEM),
        scratch_shapes=[
            pltpu.VMEM((N_DEV, 2, m_rows), jnp.float32),
            pltpu.SemaphoreType.DMA((N_DEV - 1,)),
            pltpu.SemaphoreType.DMA((N_DEV,)),
        ],
        compiler_params=pltpu.CompilerParams(collective_id=0),
    )(x)
